# baseline (speedup 1.0000x reference)
import jax
import jax.numpy as jnp
from jax import lax
from jax.experimental import pallas as pl
from jax.experimental.pallas import tpu as pltpu
from jax.experimental.pallas import tpu_sc as plsc

_SHARP = 10.0
_B, _N = 16, 20000
_BC = 640
_XW = 896
_STRIDE = 960
_OFFH = 16


def _compute(x, xm2, xm1, xp1, xp2, thresh):
    pooled = jnp.maximum(
        jnp.maximum(jnp.maximum(xm2, xm1), jnp.maximum(xp1, xp2)), x)
    ea = jnp.exp(_SHARP * (thresh - x))
    eb = jnp.exp(_SHARP * (pooled - x))
    smooth = x / ((1.0 + ea) * (1.0 + eb))
    m = smooth >= thresh
    return smooth, jnp.where(m, 1, 0), jnp.where(m, x, 0.0)


def _sc_body(pm_hbm, lt_hbm, smooth_hbm, mask_hbm, pv_hbm,
             xbuf, xtail, sbuf, mbuf, pbuf,
             sta, stb, mta, mtb, pta, ptb, xlin, ltv):
    c = lax.axis_index("c")
    s = lax.axis_index("s")
    w = s * 2 + c
    c0 = w * _BC
    t0 = jnp.clip(5 * w - 1, 0, 149)
    start = t0 * 128

    pltpu.sync_copy(lt_hbm, ltv)
    pltpu.sync_copy(pm_hbm.at[:, pl.ds(start, _XW)], xbuf)

    @pl.when(w == 31)
    def _():
        pltpu.sync_copy(pm_hbm.at[:, pl.ds(156 * 128, 32)], xtail)

    for r in range(_B):
        def _dk(k, carry, r=r):
            xlin[pl.ds(r * _STRIDE + _OFFH + k * 16, 16)] = xbuf[r, pl.ds(k * 16, 16)]
            return carry
        lax.fori_loop(0, _XW // 16, _dk, 0)

    @pl.when(w == 31)
    def _():
        for r in range(_B):
            for k in range(2):
                xlin[pl.ds(r * _STRIDE + _OFFH + _XW + k * 16, 16)] = \
                    xtail[r, pl.ds(k * 16, 16)]

    rows = lax.broadcasted_iota(jnp.int32, (16,), 0)

    @pl.when(w == 0)
    def _():
        e = rows * _STRIDE + _OFFH
        edge = plsc.load_gather(xlin, [e])
        plsc.store_scatter(xlin, [e - 1], edge)
        plsc.store_scatter(xlin, [e - 2], edge)

    @pl.when(w == 31)
    def _():
        e = rows * _STRIDE + _OFFH + _XW + 31
        edge = plsc.load_gather(xlin, [e])
        plsc.store_scatter(xlin, [e + 1], edge)
        plsc.store_scatter(xlin, [e + 2], edge)

    lt = ltv[...]
    thresh = 1.0 / (1.0 + jnp.exp(-lt))
    off = c0 - start + _OFFH

    @pl.when(w < 31)
    def _():
        for r in range(_B):
            def _cj(j, carry, r=r):
                b = r * _STRIDE + off + j * 16
                sm, mi, pv = _compute(
                    xlin[pl.ds(b, 16)],
                    xlin[pl.ds(b - 2, 16)], xlin[pl.ds(b - 1, 16)],
                    xlin[pl.ds(b + 1, 16)], xlin[pl.ds(b + 2, 16)], thresh)
                sbuf[r, pl.ds(j * 16, 16)] = sm
                mbuf[r, pl.ds(j * 16, 16)] = mi
                pbuf[r, pl.ds(j * 16, 16)] = pv
                return carry
            lax.fori_loop(0, _BC // 16, _cj, 0)
        pltpu.sync_copy(sbuf, smooth_hbm.at[:, pl.ds(c0, _BC)])
        pltpu.sync_copy(mbuf, mask_hbm.at[:, pl.ds(c0, _BC)])
        pltpu.sync_copy(pbuf, pv_hbm.at[:, pl.ds(c0, _BC)])

    @pl.when(w == 31)
    def _():
        def row_tail(r, carry):
            for j in range(10):
                b = r * _STRIDE + off + j * 16
                sm, mi, pv = _compute(
                    xlin[pl.ds(b, 16)],
                    xlin[pl.ds(b - 2, 16)], xlin[pl.ds(b - 1, 16)],
                    xlin[pl.ds(b + 1, 16)], xlin[pl.ds(b + 2, 16)], thresh)
                if j < 8:
                    sta[r, pl.ds(j * 16, 16)] = sm
                    mta[r, pl.ds(j * 16, 16)] = mi
                    pta[r, pl.ds(j * 16, 16)] = pv
                else:
                    stb[r, pl.ds(j * 16 - 128, 16)] = sm
                    mtb[r, pl.ds(j * 16 - 128, 16)] = mi
                    ptb[r, pl.ds(j * 16 - 128, 16)] = pv
            return carry

        lax.fori_loop(0, _B, row_tail, 0)
        pltpu.sync_copy(sta, smooth_hbm.at[:, pl.ds(19840, 128)])
        pltpu.sync_copy(mta, mask_hbm.at[:, pl.ds(19840, 128)])
        pltpu.sync_copy(pta, pv_hbm.at[:, pl.ds(19840, 128)])
        pltpu.sync_copy(stb, smooth_hbm.at[:, pl.ds(19968, 32)])
        pltpu.sync_copy(mtb, mask_hbm.at[:, pl.ds(19968, 32)])
        pltpu.sync_copy(ptb, pv_hbm.at[:, pl.ds(19968, 32)])


def kernel(peak_map, logit_thresh):
    lt = jnp.full((16,), logit_thresh, jnp.float32)
    f = pl.kernel(
        _sc_body,
        out_type=[
            jax.ShapeDtypeStruct((_B, _N), jnp.float32),
            jax.ShapeDtypeStruct((_B, _N), jnp.int32),
            jax.ShapeDtypeStruct((_B, _N), jnp.float32),
        ],
        mesh=plsc.VectorSubcoreMesh(core_axis_name="c", subcore_axis_name="s"),
        compiler_params=pltpu.CompilerParams(needs_layout_passes=False),
        scratch_types=[
            pltpu.VMEM((_B, _XW), jnp.float32),
            pltpu.VMEM((_B, 32), jnp.float32),
            pltpu.VMEM((_B, _BC), jnp.float32),
            pltpu.VMEM((_B, _BC), jnp.int32),
            pltpu.VMEM((_B, _BC), jnp.float32),
            pltpu.VMEM((_B, 128), jnp.float32),
            pltpu.VMEM((_B, 32), jnp.float32),
            pltpu.VMEM((_B, 128), jnp.int32),
            pltpu.VMEM((_B, 32), jnp.int32),
            pltpu.VMEM((_B, 128), jnp.float32),
            pltpu.VMEM((_B, 32), jnp.float32),
            pltpu.VMEM((_B * _STRIDE,), jnp.float32),
            pltpu.VMEM((16,), jnp.float32),
        ],
    )
    smooth, m_i32, pv = f(peak_map, lt)
    return (smooth, m_i32.astype(jnp.bool_), pv)


# speedup vs baseline: 1.4912x; 1.4912x over previous
import jax
import jax.numpy as jnp
from jax import lax
from jax.experimental import pallas as pl
from jax.experimental.pallas import tpu as pltpu
from jax.experimental.pallas import tpu_sc as plsc

_SHARP = 10.0
_B, _N = 16, 20000
_HALF = _N // 2
_PAD = 8
_CHUNK = _HALF + _PAD
_BUF = _HALF + 2 * _PAD
_NV = _HALF // 16


def _sc_body(pm_hbm, lt_hbm, smooth_hbm, mask_hbm, pv_hbm,
             xbuf, sbuf, mbuf, pbuf, ltv):
    row = lax.axis_index("s")
    half = lax.axis_index("c")
    c0 = half * _HALF
    src = half * (_HALF - _PAD)
    dst = _PAD - half * _PAD

    pltpu.sync_copy(lt_hbm, ltv)
    pltpu.sync_copy(pm_hbm.at[row, pl.ds(src, _CHUNK)], xbuf.at[pl.ds(dst, _CHUNK)])

    lanes = lax.broadcasted_iota(jnp.int32, (16,), 0)
    hv = jnp.full((16,), half, jnp.int32)
    edge_src = jnp.where(hv == 0, _PAD, _BUF - _PAD - 1)
    edge_dst = jnp.where(hv == 0, _PAD - 2, _BUF - _PAD) + lanes
    edge = plsc.load_gather(xbuf, [edge_src])
    plsc.store_scatter(xbuf, [edge_dst], edge, mask=lanes < 2)

    lt = ltv[...]
    thresh = 1.0 / (1.0 + jnp.exp(-lt))

    def body(i, carry):
        b = i * 16
        xm2 = xbuf[pl.ds(b + _PAD - 2, 16)]
        xm1 = xbuf[pl.ds(b + _PAD - 1, 16)]
        x = xbuf[pl.ds(b + _PAD, 16)]
        xp1 = xbuf[pl.ds(b + _PAD + 1, 16)]
        xp2 = xbuf[pl.ds(b + _PAD + 2, 16)]
        pooled = jnp.maximum(
            jnp.maximum(jnp.maximum(xm2, xm1), jnp.maximum(xp1, xp2)), x)
        ea = jnp.exp(_SHARP * (thresh - x))
        eb = jnp.exp(_SHARP * (pooled - x))
        smooth = x / ((1.0 + ea) * (1.0 + eb))
        m = smooth >= thresh
        sbuf[pl.ds(b, 16)] = smooth
        mbuf[pl.ds(b, 16)] = jnp.where(m, 1, 0)
        pbuf[pl.ds(b, 16)] = jnp.where(m, x, 0.0)
        return carry

    lax.fori_loop(0, _NV, body, 0)

    pltpu.sync_copy(sbuf, smooth_hbm.at[row, pl.ds(c0, _HALF)])
    pltpu.sync_copy(mbuf, mask_hbm.at[row, pl.ds(c0, _HALF)])
    pltpu.sync_copy(pbuf, pv_hbm.at[row, pl.ds(c0, _HALF)])


def kernel(peak_map, logit_thresh):
    lt = jnp.full((16,), logit_thresh, jnp.float32)
    f = pl.kernel(
        _sc_body,
        out_type=[
            jax.ShapeDtypeStruct((_B, _N), jnp.float32),
            jax.ShapeDtypeStruct((_B, _N), jnp.int32),
            jax.ShapeDtypeStruct((_B, _N), jnp.float32),
        ],
        mesh=plsc.VectorSubcoreMesh(core_axis_name="c", subcore_axis_name="s"),
        compiler_params=pltpu.CompilerParams(
            use_tc_tiling_on_sc=False, needs_layout_passes=False),
        scratch_types=[
            pltpu.VMEM((_BUF,), jnp.float32),
            pltpu.VMEM((_HALF,), jnp.float32),
            pltpu.VMEM((_HALF,), jnp.int32),
            pltpu.VMEM((_HALF,), jnp.float32),
            pltpu.VMEM((16,), jnp.float32),
        ],
    )
    smooth, m_i32, pv = f(peak_map, lt)
    return (smooth, m_i32.astype(jnp.bool_), pv)


# speedup vs baseline: 1.5091x; 1.0120x over previous
import jax
import jax.numpy as jnp
from jax import lax
from jax.experimental import pallas as pl
from jax.experimental.pallas import tpu as pltpu
from jax.experimental.pallas import tpu_sc as plsc

_SHARP = 10.0
_B, _N = 16, 20000
_HALF = _N // 2
_PAD = 8
_CHUNK = _HALF + _PAD
_BUF = _HALF + 2 * _PAD
_NV = _HALF // 16


def _sc_body(pm_hbm, lt_hbm, smooth_hbm, mask_hbm, pv_hbm,
             xbuf, sbuf, mbuf, pbuf, ltv, sem_s, sem_m, sem_p):
    row = lax.axis_index("s")
    half = lax.axis_index("c")
    c0 = half * _HALF
    src = half * (_HALF - _PAD)
    dst = _PAD - half * _PAD

    pltpu.sync_copy(lt_hbm, ltv)
    pltpu.sync_copy(pm_hbm.at[row, pl.ds(src, _CHUNK)], xbuf.at[pl.ds(dst, _CHUNK)])

    lanes = lax.broadcasted_iota(jnp.int32, (16,), 0)
    hv = jnp.full((16,), half, jnp.int32)
    edge_src = jnp.where(hv == 0, _PAD, _BUF - _PAD - 1)
    edge_dst = jnp.where(hv == 0, _PAD - 2, _BUF - _PAD) + lanes
    edge = plsc.load_gather(xbuf, [edge_src])
    plsc.store_scatter(xbuf, [edge_dst], edge, mask=lanes < 2)

    lt = ltv[...]
    thresh = 1.0 / (1.0 + jnp.exp(-lt))

    def body(i, carry):
        b = i * 16
        xm2 = xbuf[pl.ds(b + _PAD - 2, 16)]
        xm1 = xbuf[pl.ds(b + _PAD - 1, 16)]
        x = xbuf[pl.ds(b + _PAD, 16)]
        xp1 = xbuf[pl.ds(b + _PAD + 1, 16)]
        xp2 = xbuf[pl.ds(b + _PAD + 2, 16)]
        pooled = jnp.maximum(
            jnp.maximum(jnp.maximum(xm2, xm1), jnp.maximum(xp1, xp2)), x)
        ea = jnp.exp(_SHARP * (thresh - x))
        eb = jnp.exp(_SHARP * (pooled - x))
        smooth = x / ((1.0 + ea) * (1.0 + eb))
        m = smooth >= thresh
        sbuf[pl.ds(b, 16)] = smooth
        mbuf[pl.ds(b, 16)] = jnp.where(m, 1, 0)
        pbuf[pl.ds(b, 16)] = jnp.where(m, x, 0.0)
        return carry

    hc = _HALF // 2
    hv2 = _NV // 2

    lax.fori_loop(0, hv2, body, 0)
    cp_s0 = pltpu.async_copy(sbuf.at[pl.ds(0, hc)],
                             smooth_hbm.at[row, pl.ds(c0, hc)], sem_s)
    cp_m0 = pltpu.async_copy(mbuf.at[pl.ds(0, hc)],
                             mask_hbm.at[row, pl.ds(c0, hc)], sem_m)
    cp_p0 = pltpu.async_copy(pbuf.at[pl.ds(0, hc)],
                             pv_hbm.at[row, pl.ds(c0, hc)], sem_p)

    lax.fori_loop(hv2, _NV, body, 0)
    cp_s0.wait()
    cp_m0.wait()
    cp_p0.wait()
    pltpu.sync_copy(sbuf.at[pl.ds(hc, hc)], smooth_hbm.at[row, pl.ds(c0 + hc, hc)])
    pltpu.sync_copy(mbuf.at[pl.ds(hc, hc)], mask_hbm.at[row, pl.ds(c0 + hc, hc)])
    pltpu.sync_copy(pbuf.at[pl.ds(hc, hc)], pv_hbm.at[row, pl.ds(c0 + hc, hc)])


def kernel(peak_map, logit_thresh):
    lt = jnp.full((16,), logit_thresh, jnp.float32)
    f = pl.kernel(
        _sc_body,
        out_type=[
            jax.ShapeDtypeStruct((_B, _N), jnp.float32),
            jax.ShapeDtypeStruct((_B, _N), jnp.int32),
            jax.ShapeDtypeStruct((_B, _N), jnp.float32),
        ],
        mesh=plsc.VectorSubcoreMesh(core_axis_name="c", subcore_axis_name="s"),
        compiler_params=pltpu.CompilerParams(
            use_tc_tiling_on_sc=False, needs_layout_passes=False),
        scratch_types=[
            pltpu.VMEM((_BUF,), jnp.float32),
            pltpu.VMEM((_HALF,), jnp.float32),
            pltpu.VMEM((_HALF,), jnp.int32),
            pltpu.VMEM((_HALF,), jnp.float32),
            pltpu.VMEM((16,), jnp.float32),
            pltpu.SemaphoreType.DMA,
            pltpu.SemaphoreType.DMA,
            pltpu.SemaphoreType.DMA,
        ],
    )
    smooth, m_i32, pv = f(peak_map, lt)
    return (smooth, m_i32.astype(jnp.bool_), pv)
